# Initial kernel scaffold; baseline (speedup 1.0000x reference)
import jax
import jax.numpy as jnp
from jax.experimental import pallas as pl


def kernel(inputs, table):
    raise NotImplementedError("write your pallas kernel here")


# speedup vs baseline: 2.9677x; 2.9677x over previous
import functools

import jax
import jax.numpy as jnp
from jax import lax
from jax.experimental import pallas as pl
from jax.experimental.pallas import tpu as pltpu
from jax.experimental.pallas import tpu_sc as plsc

VOCAB = 100000
EMBED_DIM = 128
BATCH = 4096
SEQ = 50

NC = 2
NS = 16
NW = NC * NS
B = BATCH * SEQ
BPW = B // NW
C = 128
NCHUNK = BPW // C


def _gather_kernel(idx_hbm, table_hbm, out_hbm, idx_v, rows_v, sem):
    wid = lax.axis_index("s") * NC + lax.axis_index("c")
    base = wid * BPW
    pltpu.sync_copy(idx_hbm.at[wid], idx_v)

    def chunk(j, carry):
        pltpu.async_copy(table_hbm.at[idx_v.at[j]], rows_v, sem).wait()
        pltpu.sync_copy(rows_v, out_hbm.at[pl.ds(base + j * C, C)])
        return carry

    lax.fori_loop(0, NCHUNK, chunk, 0)


@jax.jit
def kernel(inputs, table):
    idx = inputs.reshape(NW, NCHUNK, C)
    mesh = plsc.VectorSubcoreMesh(core_axis_name="c", subcore_axis_name="s")
    out = pl.kernel(
        _gather_kernel,
        out_type=jax.ShapeDtypeStruct((B, EMBED_DIM), jnp.float32),
        mesh=mesh,
        scratch_types=[
            pltpu.VMEM((NCHUNK, C), jnp.int32),
            pltpu.VMEM((C, EMBED_DIM), jnp.float32),
            pltpu.SemaphoreType.DMA,
        ],
    )(idx, table)
    return out.reshape(BATCH, SEQ, EMBED_DIM)


# speedup vs baseline: 3.3510x; 1.1292x over previous
import functools

import jax
import jax.numpy as jnp
from jax import lax
from jax.experimental import pallas as pl
from jax.experimental.pallas import tpu as pltpu
from jax.experimental.pallas import tpu_sc as plsc

VOCAB = 100000
EMBED_DIM = 128
BATCH = 4096
SEQ = 50

NC = 2
NS = 16
NW = NC * NS
B = BATCH * SEQ
BPW = B // NW
C = 128
NCHUNK = BPW // C
NBUF = 5


def _gather_kernel(idx_hbm, table_hbm, out_hbm, idx_v, rows_v, sem_g, sem_w):
    wid = lax.axis_index("s") * NC + lax.axis_index("c")
    base = wid * BPW
    pltpu.sync_copy(idx_hbm.at[wid], idx_v)

    for b in range(NBUF):
        pltpu.async_copy(table_hbm.at[idx_v.at[b]], rows_v.at[b], sem_g.at[b])

    @pl.loop(0, NCHUNK - NBUF, step=NBUF)
    def steady(g):
        for b in range(NBUF):
            j = g + b
            out_slice = out_hbm.at[pl.ds(base + j * C, C)]
            pltpu.make_async_copy(table_hbm.at[pl.ds(0, C)], rows_v.at[b],
                                  sem_g.at[b]).wait()
            pltpu.async_copy(rows_v.at[b], out_slice, sem_w.at[b])
            pltpu.make_async_copy(rows_v.at[b], out_slice, sem_w.at[b]).wait()
            pltpu.async_copy(table_hbm.at[idx_v.at[j + NBUF]], rows_v.at[b],
                             sem_g.at[b])

    for b in range(NBUF):
        j = NCHUNK - NBUF + b
        pltpu.make_async_copy(table_hbm.at[pl.ds(0, C)], rows_v.at[b],
                              sem_g.at[b]).wait()
        pltpu.async_copy(rows_v.at[b], out_hbm.at[pl.ds(base + j * C, C)],
                         sem_w.at[b])
    for b in range(NBUF):
        j = NCHUNK - NBUF + b
        pltpu.make_async_copy(rows_v.at[b], out_hbm.at[pl.ds(base + j * C, C)],
                              sem_w.at[b]).wait()


@jax.jit
def kernel(inputs, table):
    idx = inputs.reshape(NW, NCHUNK, C)
    mesh = plsc.VectorSubcoreMesh(core_axis_name="c", subcore_axis_name="s")
    out = pl.kernel(
        _gather_kernel,
        out_type=jax.ShapeDtypeStruct((B, EMBED_DIM), jnp.float32),
        mesh=mesh,
        scratch_types=[
            pltpu.VMEM((NCHUNK, C), jnp.int32),
            pltpu.VMEM((NBUF, C, EMBED_DIM), jnp.float32),
            pltpu.SemaphoreType.DMA((NBUF,)),
            pltpu.SemaphoreType.DMA((NBUF,)),
        ],
    )(idx, table)
    return out.reshape(BATCH, SEQ, EMBED_DIM)


# speedup vs baseline: 10.4108x; 3.1068x over previous
import functools

import jax
import jax.numpy as jnp
from jax import lax
from jax.experimental import pallas as pl
from jax.experimental.pallas import tpu as pltpu
from jax.experimental.pallas import tpu_sc as plsc

VOCAB = 100000
EMBED_DIM = 128
BATCH = 4096
SEQ = 50

NC = 2
NS = 16
NW = NC * NS
B = BATCH * SEQ
BPW = B // NW
C = 128
NCHUNK = BPW // C
G = 2
NG = NCHUNK // G


def _gather_kernel(idx_hbm, table_hbm, out_hbm, idx_v, rows_v, sem_g, sem_w):
    wid = lax.axis_index("s") * NC + lax.axis_index("c")
    base = wid * BPW
    pltpu.sync_copy(idx_hbm.at[wid], idx_v)

    def issue_g(t, p):
        for k in range(G):
            pltpu.async_copy(table_hbm.at[idx_v.at[t * G + k]],
                             rows_v.at[p, pl.ds(k * C, C)], sem_g.at[p])

    def wait_g(p):
        pltpu.make_async_copy(table_hbm.at[pl.ds(0, G * C)], rows_v.at[p],
                              sem_g.at[p]).wait()

    def issue_w(t, p):
        pltpu.async_copy(rows_v.at[p],
                         out_hbm.at[pl.ds(base + t * G * C, G * C)],
                         sem_w.at[p])

    def wait_w(t, p):
        pltpu.make_async_copy(rows_v.at[p],
                              out_hbm.at[pl.ds(base + t * G * C, G * C)],
                              sem_w.at[p]).wait()

    issue_g(0, 0)
    issue_g(1, 1)
    wait_g(0)
    issue_w(0, 0)

    @pl.loop(2, NG - 1, step=2)
    def steady(t0):
        for dp in range(2):
            t = t0 + dp
            p = dp
            wait_w(t - 2, p)
            issue_g(t, p)
            wait_g(1 - p)
            issue_w(t - 1, 1 - p)

    wait_w(NG - 3, 0)
    issue_g(NG - 1, 0)
    wait_g(1)
    issue_w(NG - 2, 1)
    wait_w(NG - 2, 1)
    wait_g(0)
    issue_w(NG - 1, 0)
    wait_w(NG - 1, 0)


@jax.jit
def kernel(inputs, table):
    idx = inputs.T.reshape(NW, NCHUNK, C)
    mesh = plsc.VectorSubcoreMesh(core_axis_name="c", subcore_axis_name="s")
    out = pl.kernel(
        _gather_kernel,
        out_type=jax.ShapeDtypeStruct((B, EMBED_DIM), jnp.float32),
        mesh=mesh,
        scratch_types=[
            pltpu.VMEM((NCHUNK, C), jnp.int32),
            pltpu.VMEM((2, G * C, EMBED_DIM), jnp.float32),
            pltpu.SemaphoreType.DMA((2,)),
            pltpu.SemaphoreType.DMA((2,)),
        ],
    )(idx, table)
    return out.reshape(SEQ, BATCH, EMBED_DIM).transpose(1, 0, 2)


# speedup vs baseline: 10.4729x; 1.0060x over previous
import jax
import jax.numpy as jnp
from jax import lax
from jax.experimental import pallas as pl
from jax.experimental.pallas import tpu as pltpu
from jax.experimental.pallas import tpu_sc as plsc

VOCAB = 100000
EMBED_DIM = 128
BATCH = 4096
SEQ = 50

NC = 2
NS = 16
NW = NC * NS
B = BATCH * SEQ
BPW = B // NW
C = 128
NCHUNK = BPW // C
NBUF = 5


K = NBUF - 1


def _gather_kernel(idx_hbm, table_hbm, out_hbm, idx_v, rows_v, sem_g, sem_w):
    wid = lax.axis_index("s") * NC + lax.axis_index("c")
    base = wid * BPW
    pltpu.sync_copy(idx_hbm.at[wid], idx_v)

    def wait_g(b):
        pltpu.make_async_copy(table_hbm.at[pl.ds(0, C)], rows_v.at[b],
                              sem_g.at[b]).wait()

    def wait_w(b, j):
        pltpu.make_async_copy(rows_v.at[b],
                              out_hbm.at[pl.ds(base + j * C, C)],
                              sem_w.at[b]).wait()

    def issue_g(j, b):
        pltpu.async_copy(table_hbm.at[idx_v.at[j]], rows_v.at[b], sem_g.at[b])

    def issue_w(j, b):
        pltpu.async_copy(rows_v.at[b], out_hbm.at[pl.ds(base + j * C, C)],
                         sem_w.at[b])

    for j in range(K):
        issue_g(j, j % NBUF)
    issue_g(K, K % NBUF)
    wait_g(0)
    issue_w(0, 0)

    @pl.loop(NBUF, NCHUNK, step=NBUF)
    def steady(g):
        for b0 in range(NBUF):
            j = g + b0
            b = b0
            wait_w(b, j - NBUF)
            issue_g(j, b)
            bw = (b0 - K) % NBUF
            wait_g(bw)
            issue_w(j - K, bw)

    for j in range(NCHUNK, NCHUNK + K):
        b = j % NBUF
        wait_w(b, j - NBUF)
        bw = (j - K) % NBUF
        wait_g(bw)
        issue_w(j - K, bw)
    wait_w((NCHUNK - 1) % NBUF, NCHUNK - 1)


@jax.jit
def kernel(inputs, table):
    idx = inputs.T.reshape(NW, NCHUNK, C)
    mesh = plsc.VectorSubcoreMesh(core_axis_name="c", subcore_axis_name="s")
    out = pl.kernel(
        _gather_kernel,
        out_type=jax.ShapeDtypeStruct((B, EMBED_DIM), jnp.float32),
        mesh=mesh,
        scratch_types=[
            pltpu.VMEM((NCHUNK, C), jnp.int32),
            pltpu.VMEM((NBUF, C, EMBED_DIM), jnp.float32),
            pltpu.SemaphoreType.DMA((NBUF,)),
            pltpu.SemaphoreType.DMA((NBUF,)),
        ],
    )(idx, table)
    return out.reshape(SEQ, BATCH, EMBED_DIM).transpose(1, 0, 2)
